# baseline (speedup 1.0000x reference)
import jax
import jax.numpy as jnp
from jax.experimental import pallas as pl
from jax.experimental.pallas import tpu as pltpu


def _gcn_block(adj_ref, x_ref, w_ref, b_ref, out_ref, h_ref):
    @pl.when(pl.program_id(0) == 0)
    def _():
        h_ref[...] = jnp.dot(x_ref[...], w_ref[...],
                             preferred_element_type=jnp.float32)
    out_ref[...] = jnp.dot(adj_ref[...], h_ref[...],
                           preferred_element_type=jnp.float32) + b_ref[...]


def kernel(adj, x, W, b):
    n, k = adj.shape
    d_in = x.shape[1]
    d_out = W.shape[1]
    bm = 80
    if n % bm:
        bm = n
    grid = (n // bm,)
    out = pl.pallas_call(
        _gcn_block,
        grid=grid,
        in_specs=[
            pl.BlockSpec((bm, k), lambda i: (i, 0)),
            pl.BlockSpec((k, d_in), lambda i: (0, 0)),
            pl.BlockSpec((d_in, d_out), lambda i: (0, 0)),
            pl.BlockSpec((1, d_out), lambda i: (0, 0)),
        ],
        out_specs=pl.BlockSpec((bm, d_out), lambda i: (i, 0)),
        out_shape=jax.ShapeDtypeStruct((n, d_out), jnp.float32),
        scratch_shapes=[pltpu.VMEM((k, d_out), jnp.float32)],
    )(adj, x, W, b.reshape(1, d_out))
    return out


# speedup vs baseline: 1.3689x; 1.3689x over previous
import jax
import jax.numpy as jnp
from jax.experimental import pallas as pl
from jax.experimental.pallas import tpu as pltpu


def _gcn_block(adj_ref, x_ref, w_ref, b_ref, out_ref, h_ref):
    @pl.when(pl.program_id(0) == 0)
    def _():
        h_ref[...] = jnp.dot(x_ref[...], w_ref[...],
                             preferred_element_type=jnp.float32)
    out_ref[...] = jnp.dot(adj_ref[...], h_ref[...],
                           preferred_element_type=jnp.float32) + b_ref[...]


def kernel(adj, x, W, b):
    n, k = adj.shape
    d_in = x.shape[1]
    d_out = W.shape[1]
    bm = 200
    if n % bm:
        bm = n
    grid = (n // bm,)
    out = pl.pallas_call(
        _gcn_block,
        grid=grid,
        in_specs=[
            pl.BlockSpec((bm, k), lambda i: (i, 0)),
            pl.BlockSpec((k, d_in), lambda i: (0, 0)),
            pl.BlockSpec((d_in, d_out), lambda i: (0, 0)),
            pl.BlockSpec((1, d_out), lambda i: (0, 0)),
        ],
        out_specs=pl.BlockSpec((bm, d_out), lambda i: (i, 0)),
        out_shape=jax.ShapeDtypeStruct((n, d_out), jnp.float32),
        scratch_shapes=[pltpu.VMEM((k, d_out), jnp.float32)],
    )(adj, x, W, b.reshape(1, d_out))
    return out
